# baseline (speedup 1.0000x reference)
import functools

import jax
import jax.numpy as jnp
from jax import lax
from jax.experimental import pallas as pl
from jax.experimental.pallas import tpu as pltpu
from jax.experimental.pallas import tpu_sc as plsc

W = 1_000_000
C = 64
N = 16_384
L = 16
WIN = 128
NFULL = W // WIN
TAIL = W - NFULL * WIN
SR = W // 2

_info = plsc.get_sparse_core_info()
NC = _info.num_cores
NS = _info.num_subcores
NW = NC * NS
BPW = N // NW
SCHUNK = 256
ICH = 128


def _transpose_body(dt_hbm, tail_hbm, scr_hbm, win_v, tb_v, sem):
    wid = lax.axis_index("s") * NC + lax.axis_index("c")

    @pl.when(wid == 0)
    def _():
        pltpu.sync_copy(tail_hbm, scr_hbm.at[pl.ds(NFULL * 64, TAIL // 2)])

    chv = [jnp.mod(jnp.arange(q * L, q * L + L, dtype=jnp.int32), C)
           for q in range(WIN // L)]

    def wbody(t, carry):
        wi = wid + t * NW
        rb = pl.multiple_of(wi * WIN, WIN)
        pltpu.sync_copy(dt_hbm.at[:, pl.ds(rb, WIN)], win_v)

        def pbody(p, c):
            r0 = 2 * p
            for q in range(WIN // L):
                row = r0 + (1 if q >= 4 else 0)
                rv = jnp.full((L,), 0, jnp.int32) + row
                vals = plsc.load_gather(win_v, [chv[q], rv])
                tb_v[p, pl.ds(q * L, L)] = vals
            return c
        lax.fori_loop(0, WIN // 2, pbody, 0)

        pltpu.sync_copy(tb_v, scr_hbm.at[pl.ds(pl.multiple_of(wi * 64, 8), 64)])
        return carry

    nwin = jnp.where(wid < NFULL - (NFULL // NW) * NW, NFULL // NW + 1,
                     NFULL // NW)
    lax.fori_loop(0, nwin, wbody, 0)


@functools.partial(
    pl.kernel,
    mesh=plsc.VectorSubcoreMesh(core_axis_name="c", subcore_axis_name="s"),
    out_type=jax.ShapeDtypeStruct((SR, 2 * C), jnp.float32),
    compiler_params=pltpu.CompilerParams(needs_layout_passes=False),
    scratch_types=[
        pltpu.VMEM((C, WIN), jnp.float32),
        pltpu.VMEM((C, WIN), jnp.float32),
        pltpu.SemaphoreType.DMA,
    ],
)
def _transpose(dt_hbm, tail_hbm, scr_hbm, *scratch):
    _transpose_body(dt_hbm, tail_hbm, scr_hbm, *scratch)


def _sample_body(scr_hbm, param_hbm, out_hbm,
                 param_v, w_v, g0_v, g1_v, o0_v, o1_v, b0_v, b1_v, out_c, sem):
    wid = lax.axis_index("s") * NC + lax.axis_index("c")
    base = wid * BPW

    pltpu.sync_copy(param_hbm.at[pl.ds(base, BPW)], param_v)

    for i in range(BPW // L):
        p = param_v[pl.ds(i * L, L)]
        x = jnp.minimum(jnp.maximum(p, 0.0), 1.0) * float(W - 1)
        x0i = x.astype(jnp.int32)
        w = x - x0i.astype(jnp.float32)
        x1i = jnp.minimum(x0i + 1, W - 1)
        sl = pl.ds(i * L, L)
        w_v[sl] = w
        o0_v[sl] = (x0i & 1) << 6
        o1_v[sl] = (x1i & 1) << 6
        j = i % (ICH // L)
        g0_v[i // (ICH // L), pl.ds(j * L, L)] = x0i >> 1
        g1_v[i // (ICH // L), pl.ds(j * L, L)] = x1i >> 1

    def sbatch(s, carry):
        sbase = s * SCHUNK
        cps = []
        for j in range(SCHUNK // ICH):
            ci = pl.ds(j * ICH, ICH)
            cps.append(pltpu.make_async_copy(
                scr_hbm.at[g0_v.at[s * (SCHUNK // ICH) + j]], b0_v.at[ci], sem))
            cps.append(pltpu.make_async_copy(
                scr_hbm.at[g1_v.at[s * (SCHUNK // ICH) + j]], b1_v.at[ci], sem))
        for cp in cps:
            cp.start()
        for cp in cps:
            cp.wait()

        lane = lax.iota(jnp.int32, L)

        def grp(g, c2):
            gsl = pl.ds(sbase + g * L, L)
            w16 = w_v[gsl]
            om16 = 1.0 - w16
            o016 = o0_v[gsl]
            o116 = o1_v[gsl]
            for j in range(L):
                r = g * L + j
                rv = jnp.full((L,), 0, jnp.int32) + r
                w = w16[j]
                om = om16[j]
                l0 = lane + o016[j]
                l1 = lane + o116[j]
                for c in range(C // L):
                    d0 = plsc.load_gather(b0_v, [rv, l0 + c * L])
                    d1 = plsc.load_gather(b1_v, [rv, l1 + c * L])
                    out_c[r, pl.ds(c * L, L)] = d0 * om + d1 * w
            return c2
        lax.fori_loop(0, SCHUNK // L, grp, 0)
        pltpu.sync_copy(
            out_c, out_hbm.at[pl.ds(pl.multiple_of(base + sbase, 8), SCHUNK)])
        return carry
    lax.fori_loop(0, BPW // SCHUNK, sbatch, 0)


@functools.partial(
    pl.kernel,
    mesh=plsc.VectorSubcoreMesh(core_axis_name="c", subcore_axis_name="s"),
    out_type=jax.ShapeDtypeStruct((N, C), jnp.float32),
    compiler_params=pltpu.CompilerParams(needs_layout_passes=False),
    scratch_types=[
        pltpu.VMEM((BPW,), jnp.float32),
        pltpu.VMEM((BPW,), jnp.float32),
        pltpu.VMEM((BPW // ICH, ICH), jnp.int32),
        pltpu.VMEM((BPW // ICH, ICH), jnp.int32),
        pltpu.VMEM((BPW,), jnp.int32),
        pltpu.VMEM((BPW,), jnp.int32),
        pltpu.VMEM((SCHUNK, 2 * C), jnp.float32),
        pltpu.VMEM((SCHUNK, 2 * C), jnp.float32),
        pltpu.VMEM((SCHUNK, C), jnp.float32),
        pltpu.SemaphoreType.DMA,
    ],
)
def _sampler(scr_hbm, param_hbm, out_hbm, *scratch):
    _sample_body(scr_hbm, param_hbm, out_hbm, *scratch)


def kernel(data, param):
    dt = data.T
    tail = dt[:, NFULL * WIN:].T.reshape(TAIL // 2, 2 * C)
    scr = _transpose(dt, tail)
    return _sampler(scr, param)


# speedup vs baseline: 2.9389x; 2.9389x over previous
import functools

import jax
import jax.numpy as jnp
from jax import lax
from jax.experimental import pallas as pl
from jax.experimental.pallas import tpu as pltpu
from jax.experimental.pallas import tpu_sc as plsc

W = 1_000_000
C = 64
N = 16_384
L = 16
WIN = 128
NFULL = W // WIN
TAIL = W - NFULL * WIN
SR = W // 2

_info = plsc.get_sparse_core_info()
NC = _info.num_cores
NS = _info.num_subcores
NW = NC * NS
BPW = N // NW
SCHUNK = 256
ICH = 128


def _transpose_body(dt_hbm, tail_hbm, scr_hbm, win_v, tb_v, sem):
    wid = lax.axis_index("s") * NC + lax.axis_index("c")

    @pl.when(wid == 0)
    def _():
        pltpu.sync_copy(tail_hbm, scr_hbm.at[pl.ds(NFULL * 64, TAIL // 2)])

    chv = [jnp.mod(jnp.arange(q * L, q * L + L, dtype=jnp.int32), C)
           for q in range(WIN // L)]

    def wbody(t, carry):
        wi = wid + t * NW
        rb = pl.multiple_of(wi * WIN, WIN)
        pltpu.sync_copy(dt_hbm.at[:, pl.ds(rb, WIN)], win_v)

        def pbody(p, c):
            r0 = 2 * p
            for q in range(WIN // L):
                row = r0 + (1 if q >= 4 else 0)
                rv = jnp.full((L,), 0, jnp.int32) + row
                vals = plsc.load_gather(win_v, [chv[q], rv])
                tb_v[p, pl.ds(q * L, L)] = vals
            return c
        lax.fori_loop(0, WIN // 2, pbody, 0)

        pltpu.sync_copy(tb_v, scr_hbm.at[pl.ds(pl.multiple_of(wi * 64, 8), 64)])
        return carry

    nwin = jnp.where(wid < NFULL - (NFULL // NW) * NW, NFULL // NW + 1,
                     NFULL // NW)
    lax.fori_loop(0, nwin, wbody, 0)


@functools.partial(
    pl.kernel,
    mesh=plsc.VectorSubcoreMesh(core_axis_name="c", subcore_axis_name="s"),
    out_type=jax.ShapeDtypeStruct((SR, 2 * C), jnp.float32),
    compiler_params=pltpu.CompilerParams(needs_layout_passes=False),
    scratch_types=[
        pltpu.VMEM((C, WIN), jnp.float32),
        pltpu.VMEM((C, WIN), jnp.float32),
        pltpu.SemaphoreType.DMA,
    ],
)
def _transpose(dt_hbm, tail_hbm, scr_hbm, *scratch):
    _transpose_body(dt_hbm, tail_hbm, scr_hbm, *scratch)


def _sample_body(scr_hbm, param_hbm, out_hbm,
                 param_v, w_v, g0_v, g1_v, o0_v, o1_v, b0_v, b1_v, out_c, sem):
    wid = lax.axis_index("s") * NC + lax.axis_index("c")
    base = wid * BPW

    pltpu.sync_copy(param_hbm.at[pl.ds(base, BPW)], param_v)

    for i in range(BPW // L):
        p = param_v[pl.ds(i * L, L)]
        x = jnp.minimum(jnp.maximum(p, 0.0), 1.0) * float(W - 1)
        x0i = x.astype(jnp.int32)
        w = x - x0i.astype(jnp.float32)
        x1i = jnp.minimum(x0i + 1, W - 1)
        sl = pl.ds(i * L, L)
        w_v[sl] = w
        o0_v[sl] = (x0i & 1) << 6
        o1_v[sl] = (x1i & 1) << 6
        j = i % (ICH // L)
        g0_v[i // (ICH // L), pl.ds(j * L, L)] = x0i >> 1
        g1_v[i // (ICH // L), pl.ds(j * L, L)] = x1i >> 1

    def sbatch(s, carry):
        sbase = s * SCHUNK
        cps = []
        for j in range(SCHUNK // ICH):
            ci = pl.ds(j * ICH, ICH)
            cps.append(pltpu.make_async_copy(
                scr_hbm.at[g0_v.at[s * (SCHUNK // ICH) + j]], b0_v.at[ci], sem))
            cps.append(pltpu.make_async_copy(
                scr_hbm.at[g1_v.at[s * (SCHUNK // ICH) + j]], b1_v.at[ci], sem))
        for cp in cps:
            cp.start()
        for cp in cps:
            cp.wait()

        lane = lax.iota(jnp.int32, L)

        def grp(g, c2):
            gsl = pl.ds(sbase + g * L, L)
            w16 = w_v[gsl]
            om16 = 1.0 - w16
            o016 = o0_v[gsl]
            o116 = o1_v[gsl]
            for j in range(L):
                r = g * L + j
                rv = jnp.full((L,), 0, jnp.int32) + r
                w = w16[j]
                om = om16[j]
                l0 = lane + o016[j]
                l1 = lane + o116[j]
                for c in range(C // L):
                    d0 = plsc.load_gather(b0_v, [rv, l0 + c * L])
                    d1 = plsc.load_gather(b1_v, [rv, l1 + c * L])
                    out_c[r, pl.ds(c * L, L)] = d0 * om + d1 * w
            return c2
        lax.fori_loop(0, SCHUNK // L, grp, 0)
        pltpu.sync_copy(
            out_c, out_hbm.at[pl.ds(pl.multiple_of(base + sbase, 8), SCHUNK)])
        return carry
    lax.fori_loop(0, BPW // SCHUNK, sbatch, 0)


@functools.partial(
    pl.kernel,
    mesh=plsc.VectorSubcoreMesh(core_axis_name="c", subcore_axis_name="s"),
    out_type=jax.ShapeDtypeStruct((N, C), jnp.float32),
    compiler_params=pltpu.CompilerParams(needs_layout_passes=False),
    scratch_types=[
        pltpu.VMEM((BPW,), jnp.float32),
        pltpu.VMEM((BPW,), jnp.float32),
        pltpu.VMEM((BPW // ICH, ICH), jnp.int32),
        pltpu.VMEM((BPW // ICH, ICH), jnp.int32),
        pltpu.VMEM((BPW,), jnp.int32),
        pltpu.VMEM((BPW,), jnp.int32),
        pltpu.VMEM((SCHUNK, 2 * C), jnp.float32),
        pltpu.VMEM((SCHUNK, 2 * C), jnp.float32),
        pltpu.VMEM((SCHUNK, C), jnp.float32),
        pltpu.SemaphoreType.DMA,
    ],
)
def _sampler(scr_hbm, param_hbm, out_hbm, *scratch):
    _sample_body(scr_hbm, param_hbm, out_hbm, *scratch)


def kernel(data, param):
    scr = data.reshape(SR, 2 * C)
    return _sampler(scr, param)


# speedup vs baseline: 5.8827x; 2.0017x over previous
import functools

import jax
import jax.numpy as jnp
from jax import lax
from jax.experimental import pallas as pl
from jax.experimental.pallas import tpu as pltpu
from jax.experimental.pallas import tpu_sc as plsc

W = 1_000_000
C = 64
N = 16_384
L = 16
WIN = 128

_info = plsc.get_sparse_core_info()
NC = _info.num_cores
NS = _info.num_subcores
NW = NC * NS
BPW = N // NW
SB = 4
OT = 128


def _body(dt_hbm, param_hbm, ot_hbm,
          param_v, w_v, rl_v, rl1_v, wb_v, win_v, spare_v, out_t, sem):
    wid = lax.axis_index("s") * NC + lax.axis_index("c")
    base = wid * BPW

    pltpu.sync_copy(param_hbm.at[pl.ds(base, BPW)], param_v)

    for i in range(BPW // L):
        p = param_v[pl.ds(i * L, L)]
        x = jnp.minimum(jnp.maximum(p, 0.0), 1.0) * float(W - 1)
        x0i = x.astype(jnp.int32)
        w = x - x0i.astype(jnp.float32)
        rl = x0i & (WIN - 1)
        sl = pl.ds(i * L, L)
        w_v[sl] = w
        rl_v[sl] = rl
        rl1_v[sl] = jnp.minimum(rl + 1, WIN - 1)
        wb_v[sl] = (x0i >> 7) << 7

    lane = lax.iota(jnp.int32, L)
    sidx = lane >> 2
    cidx = lane & 3
    zero16 = jnp.full((L,), 0, jnp.int32)

    def gbody(g, carry):
        gb = g * L
        gsl = pl.ds(gb, L)
        wb16 = wb_v[gsl]
        rl16 = rl_v[gsl]
        w16 = w_v[gsl]
        om16 = 1.0 - w16

        for sbi in range(L // SB):
            sb = gb + sbi * SB
            cps = []
            for j in range(SB):
                wb = wb16[sbi * SB + j]
                cps.append(pltpu.make_async_copy(
                    dt_hbm.at[:, pl.ds(pl.multiple_of(wb, WIN), WIN)],
                    win_v.at[j], sem))
            for cp in cps:
                cp.start()
            for cp in cps:
                cp.wait()

            sv = sb + sidx
            rl4 = plsc.load_gather(rl_v, [sv])
            rl14 = plsc.load_gather(rl1_v, [sv])
            w4 = plsc.load_gather(w_v, [sv])
            om4 = 1.0 - w4
            scol = (sb % OT) + sidx
            for c0 in range(0, C, 4):
                cv = cidx + c0
                d0 = plsc.load_gather(win_v, [sidx, cv, rl4])
                d1 = plsc.load_gather(win_v, [sidx, cv, rl14])
                plsc.store_scatter(out_t, [cv, scol], d0 * om4 + d1 * w4)

            for j in range(SB):
                rl_s = rl16[sbi * SB + j]

                @pl.when(rl_s == WIN - 1)
                def _(j=j, sb=sb):
                    wb2 = wb16[sbi * SB + j] + WIN
                    pltpu.sync_copy(
                        dt_hbm.at[:, pl.ds(pl.multiple_of(wb2, WIN), WIN)],
                        spare_v)
                    wsc = w16[sbi * SB + j]
                    omc = om16[sbi * SB + j]
                    colv = jnp.full((L,), (sb % OT) + j, jnp.int32)
                    jv = jnp.full((L,), j, jnp.int32)
                    last = jnp.full((L,), WIN - 1, jnp.int32)
                    for c0 in range(0, C, L):
                        cv16 = lane + c0
                        d0 = plsc.load_gather(win_v, [jv, cv16, last])
                        d1 = plsc.load_gather(spare_v, [cv16, zero16])
                        plsc.store_scatter(out_t, [cv16, colv],
                                           d0 * omc + d1 * wsc)

        @pl.when(g % (OT // L) == (OT // L) - 1)
        def _():
            ob = base + (g // (OT // L)) * OT
            pltpu.sync_copy(
                out_t, ot_hbm.at[:, pl.ds(pl.multiple_of(ob, OT), OT)])
        return carry

    lax.fori_loop(0, BPW // L, gbody, 0)


@functools.partial(
    pl.kernel,
    mesh=plsc.VectorSubcoreMesh(core_axis_name="c", subcore_axis_name="s"),
    out_type=jax.ShapeDtypeStruct((C, N), jnp.float32),
    compiler_params=pltpu.CompilerParams(
        needs_layout_passes=False, disable_bounds_checks=True),
    scratch_types=[
        pltpu.VMEM((BPW,), jnp.float32),
        pltpu.VMEM((BPW,), jnp.float32),
        pltpu.VMEM((BPW,), jnp.int32),
        pltpu.VMEM((BPW,), jnp.int32),
        pltpu.VMEM((BPW,), jnp.int32),
        pltpu.VMEM((SB, C, WIN), jnp.float32),
        pltpu.VMEM((C, WIN), jnp.float32),
        pltpu.VMEM((C, OT), jnp.float32),
        pltpu.SemaphoreType.DMA,
    ],
)
def _sampler(dt_hbm, param_hbm, ot_hbm, *scratch):
    _body(dt_hbm, param_hbm, ot_hbm, *scratch)


def kernel(data, param):
    return _sampler(data.T, param).T


# speedup vs baseline: 9.2756x; 1.5767x over previous
import functools

import jax
import jax.numpy as jnp
from jax import lax
from jax.experimental import pallas as pl
from jax.experimental.pallas import tpu as pltpu
from jax.experimental.pallas import tpu_sc as plsc

W = 1_000_000
C = 64
N = 16_384
L = 16
WIN = 128

_info = plsc.get_sparse_core_info()
NC = _info.num_cores
NS = _info.num_subcores
NW = NC * NS
BPW = N // NW
SB = 4
OT = 128


def _body(dt_hbm, param_hbm, ot_hbm,
          param_v, w_v, rl_v, rl1_v, wb_v, win_v, spare_v, out_t, sem):
    wid = lax.axis_index("s") * NC + lax.axis_index("c")
    base = wid * BPW

    pltpu.sync_copy(param_hbm.at[pl.ds(base, BPW)], param_v)

    for i in range(BPW // L):
        p = param_v[pl.ds(i * L, L)]
        x = jnp.minimum(jnp.maximum(p, 0.0), 1.0) * float(W - 1)
        x0i = x.astype(jnp.int32)
        w = x - x0i.astype(jnp.float32)
        rl = x0i & (WIN - 1)
        sl = pl.ds(i * L, L)
        w_v[sl] = w
        rl_v[sl] = rl
        rl1_v[sl] = jnp.minimum(rl + 1, WIN - 1)
        wb_v[sl] = (x0i >> 7) << 7

    lane = lax.iota(jnp.int32, L)
    sidx = lane >> 2
    cidx = lane & 3
    zero16 = jnp.full((L,), 0, jnp.int32)
    NB = BPW // SB

    def fire(k, buf):
        wb4 = plsc.load_gather(wb_v, [k * SB + sidx])
        for j in range(SB):
            pltpu.make_async_copy(
                dt_hbm.at[:, pl.ds(pl.multiple_of(wb4[SB * j], WIN), WIN)],
                win_v.at[buf, j], sem).start()

    fire(0, 0)

    def bbody(b, carry):
        @pl.when(b + 1 < NB)
        def _():
            fire(b + 1, (b + 1) % 2)

        buf = b % 2
        for j in range(SB):
            pltpu.make_async_copy(
                dt_hbm.at[:, pl.ds(0, WIN)], win_v.at[buf, j], sem).wait()

        sb = b * SB
        sv = sb + sidx
        rl4 = plsc.load_gather(rl_v, [sv])
        rl14 = plsc.load_gather(rl1_v, [sv])
        w4 = plsc.load_gather(w_v, [sv])
        om4 = 1.0 - w4
        scol = (sb % OT) + sidx
        for c0 in range(0, C, 4):
            cv = cidx + c0
            d0 = plsc.load_gather(win_v, [jnp.full((L,), buf, jnp.int32),
                                          sidx, cv, rl4])
            d1 = plsc.load_gather(win_v, [jnp.full((L,), buf, jnp.int32),
                                          sidx, cv, rl14])
            plsc.store_scatter(out_t, [cv, scol], d0 * om4 + d1 * w4)

        for j in range(SB):
            @pl.when(rl4[SB * j] == WIN - 1)
            def _(j=j):
                wb4 = plsc.load_gather(wb_v, [sb + sidx])
                wb2 = wb4[SB * j] + WIN
                pltpu.sync_copy(
                    dt_hbm.at[:, pl.ds(pl.multiple_of(wb2, WIN), WIN)],
                    spare_v)
                wsc = w4[SB * j]
                omc = om4[SB * j]
                colv = jnp.full((L,), 0, jnp.int32) + (scol[0] + j)
                bjv = jnp.full((L,), buf, jnp.int32)
                jv = jnp.full((L,), j, jnp.int32)
                last = jnp.full((L,), WIN - 1, jnp.int32)
                for c0 in range(0, C, L):
                    cv16 = lane + c0
                    d0 = plsc.load_gather(win_v, [bjv, jv, cv16, last])
                    d1 = plsc.load_gather(spare_v, [cv16, zero16])
                    plsc.store_scatter(out_t, [cv16, colv],
                                       d0 * omc + d1 * wsc)

        @pl.when(b % (OT // SB) == (OT // SB) - 1)
        def _():
            ob = base + (b // (OT // SB)) * OT
            pltpu.sync_copy(
                out_t, ot_hbm.at[:, pl.ds(pl.multiple_of(ob, OT), OT)])
        return carry

    lax.fori_loop(0, NB, bbody, 0)


@functools.partial(
    pl.kernel,
    mesh=plsc.VectorSubcoreMesh(core_axis_name="c", subcore_axis_name="s"),
    out_type=jax.ShapeDtypeStruct((C, N), jnp.float32),
    compiler_params=pltpu.CompilerParams(
        needs_layout_passes=False, disable_bounds_checks=True),
    scratch_types=[
        pltpu.VMEM((BPW,), jnp.float32),
        pltpu.VMEM((BPW,), jnp.float32),
        pltpu.VMEM((BPW,), jnp.int32),
        pltpu.VMEM((BPW,), jnp.int32),
        pltpu.VMEM((BPW,), jnp.int32),
        pltpu.VMEM((2, SB, C, WIN), jnp.float32),
        pltpu.VMEM((C, WIN), jnp.float32),
        pltpu.VMEM((C, OT), jnp.float32),
        pltpu.SemaphoreType.DMA,
    ],
)
def _sampler(dt_hbm, param_hbm, ot_hbm, *scratch):
    _body(dt_hbm, param_hbm, ot_hbm, *scratch)


def kernel(data, param):
    return _sampler(data.T, param).T


# speedup vs baseline: 9.4398x; 1.0177x over previous
import functools

import jax
import jax.numpy as jnp
from jax import lax
from jax.experimental import pallas as pl
from jax.experimental.pallas import tpu as pltpu
from jax.experimental.pallas import tpu_sc as plsc

W = 1_000_000
C = 64
N = 16_384
L = 16
WIN = 128

_info = plsc.get_sparse_core_info()
NC = _info.num_cores
NS = _info.num_subcores
NW = NC * NS
BPW = N // NW
SB = 4
OT = 128


def _body(dt_hbm, param_hbm, ot_hbm,
          param_v, w_v, rl_v, rl1_v, wb_v, win_v, spare_v, out_t, sem):
    wid = lax.axis_index("s") * NC + lax.axis_index("c")
    base = wid * BPW

    pltpu.sync_copy(param_hbm.at[pl.ds(base, BPW)], param_v)

    for i in range(BPW // L):
        p = param_v[pl.ds(i * L, L)]
        x = jnp.minimum(jnp.maximum(p, 0.0), 1.0) * float(W - 1)
        x0i = x.astype(jnp.int32)
        w = x - x0i.astype(jnp.float32)
        rl = x0i & (WIN - 1)
        sl = pl.ds(i * L, L)
        w_v[sl] = w
        rl_v[sl] = rl
        rl1_v[sl] = jnp.minimum(rl + 1, WIN - 1)
        wb_v[sl] = (x0i >> 7) << 7

    lane = lax.iota(jnp.int32, L)
    sidx = lane >> 2
    cidx = lane & 3
    zero16 = jnp.full((L,), 0, jnp.int32)
    NB = BPW // SB

    def fire(k, buf):
        wb4 = plsc.load_gather(wb_v, [k * SB + sidx])
        for j in range(SB):
            pltpu.make_async_copy(
                dt_hbm.at[:, pl.ds(pl.multiple_of(wb4[SB * j], WIN), WIN)],
                win_v.at[buf, j], sem).start()

    fire(0, 0)
    fire(1, 1)

    def bbody(b, carry):
        @pl.when(b + 2 < NB)
        def _():
            fire(b + 2, (b + 2) % 3)

        buf = b % 3
        for j in range(SB):
            pltpu.make_async_copy(
                dt_hbm.at[:, pl.ds(0, WIN)], win_v.at[buf, j], sem).wait()

        sb = b * SB
        sv = sb + sidx
        rl4 = plsc.load_gather(rl_v, [sv])
        rl14 = plsc.load_gather(rl1_v, [sv])
        w4 = plsc.load_gather(w_v, [sv])
        om4 = 1.0 - w4
        scol = (sb % OT) + sidx
        for c0 in range(0, C, 4):
            cv = cidx + c0
            d0 = plsc.load_gather(win_v, [jnp.full((L,), buf, jnp.int32),
                                          sidx, cv, rl4])
            d1 = plsc.load_gather(win_v, [jnp.full((L,), buf, jnp.int32),
                                          sidx, cv, rl14])
            plsc.store_scatter(out_t, [cv, scol], d0 * om4 + d1 * w4)

        for j in range(SB):
            @pl.when(rl4[SB * j] == WIN - 1)
            def _(j=j):
                wb4 = plsc.load_gather(wb_v, [sb + sidx])
                wb2 = wb4[SB * j] + WIN
                pltpu.sync_copy(
                    dt_hbm.at[:, pl.ds(pl.multiple_of(wb2, WIN), WIN)],
                    spare_v)
                wsc = w4[SB * j]
                omc = om4[SB * j]
                colv = jnp.full((L,), 0, jnp.int32) + (scol[0] + j)
                bjv = jnp.full((L,), buf, jnp.int32)
                jv = jnp.full((L,), j, jnp.int32)
                last = jnp.full((L,), WIN - 1, jnp.int32)
                for c0 in range(0, C, L):
                    cv16 = lane + c0
                    d0 = plsc.load_gather(win_v, [bjv, jv, cv16, last])
                    d1 = plsc.load_gather(spare_v, [cv16, zero16])
                    plsc.store_scatter(out_t, [cv16, colv],
                                       d0 * omc + d1 * wsc)

        @pl.when(b % (OT // SB) == (OT // SB) - 1)
        def _():
            ob = base + (b // (OT // SB)) * OT
            pltpu.sync_copy(
                out_t, ot_hbm.at[:, pl.ds(pl.multiple_of(ob, OT), OT)])
        return carry

    lax.fori_loop(0, NB, bbody, 0)


@functools.partial(
    pl.kernel,
    mesh=plsc.VectorSubcoreMesh(core_axis_name="c", subcore_axis_name="s"),
    out_type=jax.ShapeDtypeStruct((C, N), jnp.float32),
    compiler_params=pltpu.CompilerParams(
        needs_layout_passes=False, disable_bounds_checks=True),
    scratch_types=[
        pltpu.VMEM((BPW,), jnp.float32),
        pltpu.VMEM((BPW,), jnp.float32),
        pltpu.VMEM((BPW,), jnp.int32),
        pltpu.VMEM((BPW,), jnp.int32),
        pltpu.VMEM((BPW,), jnp.int32),
        pltpu.VMEM((3, SB, C, WIN), jnp.float32),
        pltpu.VMEM((C, WIN), jnp.float32),
        pltpu.VMEM((C, OT), jnp.float32),
        pltpu.SemaphoreType.DMA,
    ],
)
def _sampler(dt_hbm, param_hbm, ot_hbm, *scratch):
    _body(dt_hbm, param_hbm, ot_hbm, *scratch)


def kernel(data, param):
    return _sampler(data.T, param).T
